# baseline (speedup 1.0000x reference)
import functools

import jax
import jax.numpy as jnp
from jax import lax
from jax.experimental import pallas as pl
from jax.experimental.pallas import tpu as pltpu
from jax.experimental.pallas import tpu_sc as plsc

D_IN = 128
D_HID = 128
D_OUT = 349
N1 = 4000
N2 = 1000
NUM_ET = 7
NUM_NT = 4
NPAPER = 20000

W = 144
NSEG = NUM_ET * N2
SEG_PAD = 7040
DUMP = 7008
NC = 2
NS = 16
NW = NC * NS
CH = 128
ROWS_A = 4096
NXOUT = 1024
PER_TILE_SEG = SEG_PAD // NS
BLK = 512
RSHIFT = 15

_mesh = plsc.VectorSubcoreMesh(core_axis_name="c", subcore_axis_name="s")
_sc_params = pltpu.CompilerParams(needs_layout_passes=False,
                                  use_tc_tiling_on_sc=False)


def _agg_kernel(eb, blk, translate, *args):
    if translate:
        (xp_hbm, emb_hbm, src_hbm, dst_hbm, et_hbm, nid_hbm, nt_hbm, li_hbm,
         acc_out, xaug_out, nt_out,
         nid128, ntv, liv, gv, embv, gbuf,
         srcblk0, dstblk0, etblk0, srcblk1, dstblk1, etblk1, packc, segb,
         rows0, rows1, acc, gsem, ssem, isem0, isem1) = args
        rowsrc = xaug_out.at[lax.axis_index("c")]
    else:
        (rowsrc_hbm, src_hbm, dst_hbm, et_hbm, acc_out,
         srcblk0, dstblk0, etblk0, srcblk1, dstblk1, etblk1, packc, segb,
         rows0, rows1, acc, gsem, ssem, isem0, isem1) = args
        rowsrc = rowsrc_hbm

    cid = lax.axis_index("c")
    sid = lax.axis_index("s")
    wid = sid * NC + cid
    packs = ((eb + CH - 1) // CH) * CH

    def zrow(r, carry):
        for c in range(W // 16):
            rows0[r, pl.ds(c * 16, 16)] = jnp.zeros((16,), jnp.float32)
        return carry

    lax.fori_loop(0, CH, zrow, 0)
    base = sid * PER_TILE_SEG
    for i in range(PER_TILE_SEG // CH):
        pltpu.sync_copy(rows0, acc.at[pl.ds(base + i * CH, CH)])
    rem = PER_TILE_SEG % CH
    if rem:
        pltpu.sync_copy(rows0.at[pl.ds(0, rem)],
                        acc.at[pl.ds(base + (PER_TILE_SEG // CH) * CH, rem)])

    if translate:
        share = ROWS_A // NS
        PC = 64
        NPC = share // PC
        sems = (isem0, isem1)
        pltpu.sync_copy(emb_hbm, embv)
        pltpu.sync_copy(nid_hbm.at[pl.ds(sid * share, share)], nid128)

        def issue_ids(c):
            pltpu.async_copy(nt_hbm.at[nid128.at[pl.ds(c * PC, PC)]],
                             ntv.at[pl.ds(c * PC, PC)], sems[c % 2])
            pltpu.async_copy(li_hbm.at[nid128.at[pl.ds(c * PC, PC)]],
                             liv.at[pl.ds(c * PC, PC)], sems[c % 2])

        issue_ids(0)
        onescol = jnp.where(lax.iota(jnp.int32, 16) == 0,
                            jnp.full((16,), 1.0, jnp.float32),
                            jnp.zeros((16,), jnp.float32))
        for c in range(NPC):
            gb = sid * share + c * PC
            for _ in range(2):
                pltpu.make_async_copy(nt_hbm.at[nid128.at[pl.ds(0, PC)]],
                                      ntv.at[pl.ds(0, PC)],
                                      sems[c % 2]).wait()
            if c + 1 < NPC:
                issue_ids(c + 1)

            def ggrp(j, carry):
                lig = jnp.clip(liv[pl.ds(c * PC + j * 16, 16)], 0, NPAPER - 1)
                gv[pl.ds(j * 16, 16)] = lig
                return carry

            lax.fori_loop(0, PC // 16, ggrp, 0)

            pltpu.async_copy(xp_hbm.at[gv.at[pl.ds(0, PC)]], gbuf,
                             gsem).wait()
            rbuf = rows1 if c % 2 == 0 else rows0
            if c >= 2:
                pltpu.make_async_copy(rows0.at[pl.ds(0, PC)],
                                      xaug_out.at[cid].at[pl.ds(0, PC)],
                                      ssem).wait()

            def prow(r, carry):
                ntr = ntv[pl.ds(c * PC + r, 16)][0]

                @pl.when(ntr == 0)
                def _():
                    for q in range(D_IN // 16):
                        rbuf[r, pl.ds(q * 16, 16)] = gbuf[r, pl.ds(q * 16, 16)]

                @pl.when(ntr != 0)
                def _():
                    for q in range(D_IN // 16):
                        rbuf[r, pl.ds(q * 16, 16)] = embv[
                            ntr - 1, pl.ds(q * 16, 16)]

                rbuf[r, pl.ds(D_IN, 16)] = onescol
                return carry

            lax.fori_loop(0, PC, prow, 0)
            pltpu.async_copy(rbuf.at[pl.ds(0, PC)],
                             xaug_out.at[cid].at[pl.ds(gb, PC)], ssem)

            @pl.when(jnp.logical_and(cid == 0, gb < NXOUT))
            def _():
                pltpu.sync_copy(ntv.at[pl.ds(c * PC, PC)],
                                nt_out.at[pl.ds(gb, PC)])

        for c in range(2):
            pltpu.make_async_copy(rows0.at[pl.ds(0, PC)],
                                  xaug_out.at[cid].at[pl.ds(0, PC)],
                                  ssem).wait()

    plsc.subcore_barrier()

    nblk = eb // blk
    idb = ((srcblk0, dstblk0, etblk0, isem0), (srcblk1, dstblk1, etblk1, isem1))

    def issue_blk(b, bufs):
        off = wid * eb + b * blk
        sb, db, tb, sem = bufs
        pltpu.async_copy(src_hbm.at[pl.ds(off, blk)], sb, sem)
        pltpu.async_copy(dst_hbm.at[pl.ds(off, blk)], db, sem)
        pltpu.async_copy(et_hbm.at[pl.ds(off, blk)], tb, sem)

    def wait_blk(bufs):
        sb, db, tb, sem = bufs
        for buf in (sb, db, tb):
            pltpu.make_async_copy(src_hbm.at[pl.ds(0, blk)], buf, sem).wait()

    issue_blk(0, idb[0])

    def blk_pair(j, ptr):
        for par in range(2):
            b = 2 * j + par
            bufs = idb[par]
            nbufs = idb[1 - par]
            wait_blk(bufs)

            @pl.when(b + 1 < nblk)
            def _():
                issue_blk(b + 1, nbufs)

            sb, db, tb, _sem = bufs

            def grp(g, q):
                dv = db[pl.ds(g * 16, 16)]
                ev = tb[pl.ds(g * 16, 16)]
                sv = sb[pl.ds(g * 16, 16)]
                m = dv < N2
                packed = jnp.bitwise_or(
                    jnp.left_shift(ev * N2 + dv, RSHIFT), sv)
                plsc.store_compressed(packc.at[pl.ds(q, 16)], packed, mask=m)
                cnt = plsc.all_reduce_population_count(m)
                return q + cnt[0]

            ptr = lax.fori_loop(0, blk // 16, grp, ptr)
        return ptr

    n = lax.fori_loop(0, nblk // 2, blk_pair, jnp.int32(0))
    if nblk % 2:
        bufs = idb[(nblk - 1) % 2]
        sb, db, tb, _sem = bufs
        wait_blk(bufs)

        def tgrp(g, q):
            dv = db[pl.ds(g * 16, 16)]
            ev = tb[pl.ds(g * 16, 16)]
            sv = sb[pl.ds(g * 16, 16)]
            m = dv < N2
            packed = jnp.bitwise_or(
                jnp.left_shift(ev * N2 + dv, RSHIFT), sv)
            plsc.store_compressed(packc.at[pl.ds(q, 16)], packed, mask=m)
            cnt = plsc.all_reduce_population_count(m)
            return q + cnt[0]

        n = lax.fori_loop(0, blk // 16, tgrp, n)
    nck = (n + CH - 1) // CH

    padv = jnp.full((16,), (DUMP + sid) << RSHIFT, jnp.int32)
    iota16 = lax.iota(jnp.int32, 16)
    for j in range(CH // 16):
        off = n + j * 16
        b16 = jnp.minimum(off, packs - 16)
        cur = packc[pl.ds(b16, 16)]
        lanes = b16 + iota16
        packc[pl.ds(b16, 16)] = jnp.where(lanes < n, cur, padv)

    def rp(g, carry):
        pv = packc[pl.ds(g * 16, 16)]
        row = g // (CH // 16)
        col = (g % (CH // 16)) * 16
        segb[row, pl.ds(col, 16)] = jnp.right_shift(pv, RSHIFT)
        packc[pl.ds(g * 16, 16)] = jnp.bitwise_and(pv, (1 << RSHIFT) - 1)
        return carry

    lax.fori_loop(0, nck * (CH // 16), rp, 0)

    def issue_gather(k, buf):
        pltpu.async_copy(rowsrc.at[packc.at[pl.ds(k * CH, CH)]], buf, gsem)

    def wait_gather(buf):
        pltpu.make_async_copy(rowsrc.at[packc.at[pl.ds(0, CH)]], buf,
                              gsem).wait()

    def wait_scatter():
        pltpu.make_async_copy(rows0, acc.at[segb.at[0]], ssem).wait()

    @pl.when(nck > 0)
    def _():
        issue_gather(0, rows0)

    def outer(j, carry):
        for b in range(2):
            k = 2 * j + b
            rb = rows0 if b == 0 else rows1
            ro = rows1 if b == 0 else rows0

            @pl.when(k < nck)
            def _():
                wait_gather(rb)

                @pl.when(k >= 1)
                def _():
                    wait_scatter()

                @pl.when(k + 1 < nck)
                def _():
                    issue_gather(k + 1, ro)

                pltpu.async_copy(rb, acc.at[segb.at[k]], ssem, add=True)

        return carry

    lax.fori_loop(0, (nck + 1) // 2, outer, 0)

    @pl.when(nck > 0)
    def _():
        wait_scatter()

    plsc.subcore_barrier()

    s = sid * PER_TILE_SEG
    pltpu.sync_copy(acc.at[pl.ds(s, PER_TILE_SEG)],
                    acc_out.at[cid].at[pl.ds(s, PER_TILE_SEG)])


def _make_l0_call(eb, blk):
    packs = ((eb + CH - 1) // CH) * CH
    return functools.partial(
        pl.kernel,
        out_type=(
            jax.ShapeDtypeStruct((NC, SEG_PAD, W), jnp.float32),
            jax.ShapeDtypeStruct((NC, ROWS_A, W), jnp.float32),
            jax.ShapeDtypeStruct((NXOUT,), jnp.int32),
        ),
        mesh=_mesh,
        scratch_types=[
            pltpu.VMEM((256,), jnp.int32),
            pltpu.VMEM((272,), jnp.int32),
            pltpu.VMEM((256,), jnp.int32),
            pltpu.VMEM((128,), jnp.int32),
            pltpu.VMEM((3, D_IN), jnp.float32),
            pltpu.VMEM((64, D_IN), jnp.float32),
            pltpu.VMEM((blk,), jnp.int32),
            pltpu.VMEM((blk,), jnp.int32),
            pltpu.VMEM((blk,), jnp.int32),
            pltpu.VMEM((blk,), jnp.int32),
            pltpu.VMEM((blk,), jnp.int32),
            pltpu.VMEM((blk,), jnp.int32),
            pltpu.VMEM((packs,), jnp.int32),
            pltpu.VMEM((packs // CH, CH), jnp.int32),
            pltpu.VMEM((CH, W), jnp.float32),
            pltpu.VMEM((CH, W), jnp.float32),
            pltpu.VMEM_SHARED((SEG_PAD, W), jnp.float32),
            pltpu.SemaphoreType.DMA,
            pltpu.SemaphoreType.DMA,
            pltpu.SemaphoreType.DMA,
            pltpu.SemaphoreType.DMA,
        ],
        compiler_params=_sc_params,
    )(functools.partial(_agg_kernel, eb, blk, True))


def _make_l1_call(eb, blk):
    packs = ((eb + CH - 1) // CH) * CH
    return functools.partial(
        pl.kernel,
        out_type=jax.ShapeDtypeStruct((NC, SEG_PAD, W), jnp.float32),
        mesh=_mesh,
        scratch_types=[
            pltpu.VMEM((blk,), jnp.int32),
            pltpu.VMEM((blk,), jnp.int32),
            pltpu.VMEM((blk,), jnp.int32),
            pltpu.VMEM((blk,), jnp.int32),
            pltpu.VMEM((blk,), jnp.int32),
            pltpu.VMEM((blk,), jnp.int32),
            pltpu.VMEM((packs,), jnp.int32),
            pltpu.VMEM((packs // CH, CH), jnp.int32),
            pltpu.VMEM((CH, W), jnp.float32),
            pltpu.VMEM((CH, W), jnp.float32),
            pltpu.VMEM_SHARED((SEG_PAD, W), jnp.float32),
            pltpu.SemaphoreType.DMA,
            pltpu.SemaphoreType.DMA,
            pltpu.SemaphoreType.DMA,
            pltpu.SemaphoreType.DMA,
        ],
        compiler_params=_sc_params,
    )(functools.partial(_agg_kernel, eb, blk, False))


def _dense0_kernel(acc_ref, xaug_ref, nt_ref, w_ref, rw_ref, rb_ref, out_ref):
    acc = acc_ref[0] + acc_ref[1]
    feats = acc[:NSEG, :D_IN]
    cnt = acc[:NSEG, D_IN:D_IN + 1]
    scaled = feats / jnp.maximum(cnt, 1.0)
    h = jnp.zeros((N2, D_HID), jnp.float32)
    for i in range(NUM_ET):
        h = h + jnp.dot(scaled[i * N2:(i + 1) * N2], w_ref[i].T,
                        preferred_element_type=jnp.float32)
    x_tgt = xaug_ref[:N2, :D_IN]
    nt = nt_ref[:N2]
    for t in range(NUM_NT):
        r = jnp.dot(x_tgt, rw_ref[t].T, preferred_element_type=jnp.float32)
        r = r + rb_ref[t][None, :]
        h = h + jnp.where(nt == t, r, 0.0)
    h = jnp.maximum(h, 0.0)
    out_ref[:, :D_IN] = h
    colpad = lax.broadcasted_iota(jnp.int32, (N2, W - D_IN), 1)
    out_ref[:, D_IN:] = jnp.where(colpad == 0, 1.0, 0.0)


def _dense1_kernel(acc_ref, xaug_ref, nt_ref, w_ref, rw_ref, rb_ref, out_ref):
    acc = acc_ref[0] + acc_ref[1]
    feats = acc[:NSEG, :D_HID]
    cnt = acc[:NSEG, D_HID:D_HID + 1]
    scaled = feats / jnp.maximum(cnt, 1.0)
    o = jnp.zeros((N2, D_OUT), jnp.float32)
    for i in range(NUM_ET):
        o = o + jnp.dot(scaled[i * N2:(i + 1) * N2], w_ref[i].T,
                        preferred_element_type=jnp.float32)
    x_tgt = xaug_ref[:N2, :D_HID]
    nt = nt_ref[:N2]
    for t in range(NUM_NT):
        r = jnp.dot(x_tgt, rw_ref[t].T, preferred_element_type=jnp.float32)
        r = r + rb_ref[t][None, :]
        o = o + jnp.where(nt == t, r, 0.0)
    m = jnp.max(o, axis=1, keepdims=True)
    e = jnp.exp(o - m)
    lse = jnp.log(jnp.sum(e, axis=1, keepdims=True))
    out_ref[...] = o - m - lse


def kernel(n_id, x_paper, edge_index_0, edge_type_0, edge_index_1, edge_type_1,
           node_type, local_node_idx, emb, rel_W0, root_W0, root_b0,
           rel_W1, root_W1, root_b1):
    acc0, xaug, nt1024 = _make_l0_call(256000 // NW, 800)(
        x_paper, emb, edge_index_0[0].astype(jnp.int32),
        edge_index_0[1].astype(jnp.int32), edge_type_0.astype(jnp.int32),
        n_id.astype(jnp.int32), node_type.astype(jnp.int32),
        local_node_idx.astype(jnp.int32))

    nt1000 = nt1024[:N2][:, None]
    h1aug = pl.pallas_call(
        _dense0_kernel,
        out_shape=jax.ShapeDtypeStruct((N2, W), jnp.float32),
    )(acc0, xaug[0, :N2], nt1000, rel_W0, root_W0, root_b0)

    acc1 = _make_l1_call(64000 // NW, 400)(
        h1aug, edge_index_1[0].astype(jnp.int32),
        edge_index_1[1].astype(jnp.int32), edge_type_1.astype(jnp.int32))

    out = pl.pallas_call(
        _dense1_kernel,
        out_shape=jax.ShapeDtypeStruct((N2, D_OUT), jnp.float32),
    )(acc1, h1aug, nt1000, rel_W1, root_W1, root_b1)
    return out


# speedup vs baseline: 1.2141x; 1.2141x over previous
import functools

import jax
import jax.numpy as jnp
from jax import lax
from jax.experimental import pallas as pl
from jax.experimental.pallas import tpu as pltpu
from jax.experimental.pallas import tpu_sc as plsc

D_IN = 128
D_HID = 128
D_OUT = 349
N1 = 4000
N2 = 1000
NUM_ET = 7
NUM_NT = 4
NPAPER = 20000

W = 144
NSEG = NUM_ET * N2
SEG_PAD = 7040
DUMP = 7008
NC = 2
NS = 16
NW = NC * NS
CH = 128
ROWS_A = 4096
NXOUT = 1024
PER_TILE_SEG = SEG_PAD // NS
BLK = 512
RSHIFT = 15

_mesh = plsc.VectorSubcoreMesh(core_axis_name="c", subcore_axis_name="s")
_sc_params = pltpu.CompilerParams(needs_layout_passes=False,
                                  use_tc_tiling_on_sc=False)


def _agg_kernel(eb, blk, translate, *args):
    if translate:
        (xp_hbm, emb_hbm, src_hbm, dst_hbm, et_hbm, nid_hbm, nt_hbm, li_hbm,
         acc_out, xaug_out, nt_out,
         nid128, ntv, liv, gv, embv, gbuf,
         srcblk0, dstblk0, etblk0, srcblk1, dstblk1, etblk1, packc, segb,
         rows0, rows1, acc, gsem, ssem, isem0, isem1) = args
        rowsrc = xaug_out.at[lax.axis_index("c")]
    else:
        (rowsrc_hbm, src_hbm, dst_hbm, et_hbm, acc_out,
         srcblk0, dstblk0, etblk0, srcblk1, dstblk1, etblk1, packc, segb,
         rows0, rows1, acc, gsem, ssem, isem0, isem1) = args
        rowsrc = rowsrc_hbm

    cid = lax.axis_index("c")
    sid = lax.axis_index("s")
    wid = sid * NC + cid
    packs = ((eb + CH - 1) // CH) * CH

    def zrow(r, carry):
        for c in range(W // 16):
            rows0[r, pl.ds(c * 16, 16)] = jnp.zeros((16,), jnp.float32)
        return carry

    lax.fori_loop(0, CH, zrow, 0)
    base = sid * PER_TILE_SEG
    for i in range(PER_TILE_SEG // CH):
        pltpu.sync_copy(rows0, acc.at[pl.ds(base + i * CH, CH)])
    rem = PER_TILE_SEG % CH
    if rem:
        pltpu.sync_copy(rows0.at[pl.ds(0, rem)],
                        acc.at[pl.ds(base + (PER_TILE_SEG // CH) * CH, rem)])

    if translate:
        share = ROWS_A // NS
        PC = 64
        NPC = share // PC
        sems = (isem0, isem1)
        pltpu.sync_copy(emb_hbm, embv)
        pltpu.sync_copy(nid_hbm.at[pl.ds(sid * share, share)], nid128)

        def issue_ids(c):
            pltpu.async_copy(nt_hbm.at[nid128.at[pl.ds(c * PC, PC)]],
                             ntv.at[pl.ds(c * PC, PC)], sems[c % 2])
            pltpu.async_copy(li_hbm.at[nid128.at[pl.ds(c * PC, PC)]],
                             liv.at[pl.ds(c * PC, PC)], sems[c % 2])

        issue_ids(0)
        onescol = jnp.where(lax.iota(jnp.int32, 16) == 0,
                            jnp.full((16,), 1.0, jnp.float32),
                            jnp.zeros((16,), jnp.float32))
        for c in range(NPC):
            gb = sid * share + c * PC
            for _ in range(2):
                pltpu.make_async_copy(nt_hbm.at[nid128.at[pl.ds(0, PC)]],
                                      ntv.at[pl.ds(0, PC)],
                                      sems[c % 2]).wait()
            if c + 1 < NPC:
                issue_ids(c + 1)

            def ggrp(j, carry):
                lig = jnp.clip(liv[pl.ds(c * PC + j * 16, 16)], 0, NPAPER - 1)
                gv[pl.ds(j * 16, 16)] = lig
                return carry

            lax.fori_loop(0, PC // 16, ggrp, 0)

            pltpu.async_copy(xp_hbm.at[gv.at[pl.ds(0, PC)]], gbuf,
                             gsem).wait()
            rbuf = rows1 if c % 2 == 0 else rows0
            if c >= 2:
                pltpu.make_async_copy(rows0.at[pl.ds(0, PC)],
                                      xaug_out.at[cid].at[pl.ds(0, PC)],
                                      ssem).wait()

            def prow(r, carry):
                ntr = ntv[pl.ds(c * PC + r, 16)][0]

                @pl.when(ntr == 0)
                def _():
                    for q in range(D_IN // 16):
                        rbuf[r, pl.ds(q * 16, 16)] = gbuf[r, pl.ds(q * 16, 16)]

                @pl.when(ntr != 0)
                def _():
                    for q in range(D_IN // 16):
                        rbuf[r, pl.ds(q * 16, 16)] = embv[
                            ntr - 1, pl.ds(q * 16, 16)]

                rbuf[r, pl.ds(D_IN, 16)] = onescol
                return carry

            lax.fori_loop(0, PC, prow, 0)
            pltpu.async_copy(rbuf.at[pl.ds(0, PC)],
                             xaug_out.at[cid].at[pl.ds(gb, PC)], ssem)

            @pl.when(jnp.logical_and(cid == 0, gb < NXOUT))
            def _():
                pltpu.sync_copy(ntv.at[pl.ds(c * PC, PC)],
                                nt_out.at[pl.ds(gb, PC)])

        for c in range(2):
            pltpu.make_async_copy(rows0.at[pl.ds(0, PC)],
                                  xaug_out.at[cid].at[pl.ds(0, PC)],
                                  ssem).wait()

    plsc.subcore_barrier()

    nblk = eb // blk
    idb = ((srcblk0, dstblk0, etblk0, isem0), (srcblk1, dstblk1, etblk1, isem1))

    def issue_blk(b, bufs):
        off = wid * eb + b * blk
        sb, db, tb, sem = bufs
        pltpu.async_copy(src_hbm.at[pl.ds(off, blk)], sb, sem)
        pltpu.async_copy(dst_hbm.at[pl.ds(off, blk)], db, sem)
        pltpu.async_copy(et_hbm.at[pl.ds(off, blk)], tb, sem)

    def wait_blk(bufs):
        sb, db, tb, sem = bufs
        for buf in (sb, db, tb):
            pltpu.make_async_copy(src_hbm.at[pl.ds(0, blk)], buf, sem).wait()

    issue_blk(0, idb[0])

    def blk_pair(j, ptr):
        for par in range(2):
            b = 2 * j + par
            bufs = idb[par]
            nbufs = idb[1 - par]
            wait_blk(bufs)

            @pl.when(b + 1 < nblk)
            def _():
                issue_blk(b + 1, nbufs)

            sb, db, tb, _sem = bufs

            def grp(g, q):
                dv = db[pl.ds(g * 16, 16)]
                ev = tb[pl.ds(g * 16, 16)]
                sv = sb[pl.ds(g * 16, 16)]
                m = dv < N2
                packed = jnp.bitwise_or(
                    jnp.left_shift(ev * N2 + dv, RSHIFT), sv)
                plsc.store_compressed(packc.at[pl.ds(q, 16)], packed, mask=m)
                cnt = plsc.all_reduce_population_count(m)
                return q + cnt[0]

            ptr = lax.fori_loop(0, blk // 16, grp, ptr)
        return ptr

    n = lax.fori_loop(0, nblk // 2, blk_pair, jnp.int32(0))
    if nblk % 2:
        bufs = idb[(nblk - 1) % 2]
        sb, db, tb, _sem = bufs
        wait_blk(bufs)

        def tgrp(g, q):
            dv = db[pl.ds(g * 16, 16)]
            ev = tb[pl.ds(g * 16, 16)]
            sv = sb[pl.ds(g * 16, 16)]
            m = dv < N2
            packed = jnp.bitwise_or(
                jnp.left_shift(ev * N2 + dv, RSHIFT), sv)
            plsc.store_compressed(packc.at[pl.ds(q, 16)], packed, mask=m)
            cnt = plsc.all_reduce_population_count(m)
            return q + cnt[0]

        n = lax.fori_loop(0, blk // 16, tgrp, n)
    nck = (n + CH - 1) // CH

    padv = jnp.full((16,), (DUMP + sid) << RSHIFT, jnp.int32)
    iota16 = lax.iota(jnp.int32, 16)
    for j in range(CH // 16):
        off = n + j * 16
        b16 = jnp.minimum(off, packs - 16)
        cur = packc[pl.ds(b16, 16)]
        lanes = b16 + iota16
        packc[pl.ds(b16, 16)] = jnp.where(lanes < n, cur, padv)

    def rp(g, carry):
        pv = packc[pl.ds(g * 16, 16)]
        row = g // (CH // 16)
        col = (g % (CH // 16)) * 16
        segb[row, pl.ds(col, 16)] = jnp.right_shift(pv, RSHIFT)
        packc[pl.ds(g * 16, 16)] = jnp.bitwise_and(pv, (1 << RSHIFT) - 1)
        return carry

    lax.fori_loop(0, nck * (CH // 16), rp, 0)

    def issue_gather(k, buf):
        pltpu.async_copy(rowsrc.at[packc.at[pl.ds(k * CH, CH)]], buf, gsem)

    def wait_gather(buf):
        pltpu.make_async_copy(rowsrc.at[packc.at[pl.ds(0, CH)]], buf,
                              gsem).wait()

    def wait_scatter():
        pltpu.make_async_copy(rows0, acc.at[segb.at[0]], ssem).wait()

    @pl.when(nck > 0)
    def _():
        issue_gather(0, rows0)

    def outer(j, carry):
        for b in range(2):
            k = 2 * j + b
            rb = rows0 if b == 0 else rows1
            ro = rows1 if b == 0 else rows0

            @pl.when(k < nck)
            def _():
                wait_gather(rb)

                @pl.when(k >= 1)
                def _():
                    wait_scatter()

                @pl.when(k + 1 < nck)
                def _():
                    issue_gather(k + 1, ro)

                pltpu.async_copy(rb, acc.at[segb.at[k]], ssem, add=True)

        return carry

    lax.fori_loop(0, (nck + 1) // 2, outer, 0)

    @pl.when(nck > 0)
    def _():
        wait_scatter()

    plsc.subcore_barrier()

    s = sid * PER_TILE_SEG
    pltpu.sync_copy(acc.at[pl.ds(s, PER_TILE_SEG)],
                    acc_out.at[cid].at[pl.ds(s, PER_TILE_SEG)])


def _make_l0_call(eb, blk):
    packs = ((eb + CH - 1) // CH) * CH
    return functools.partial(
        pl.kernel,
        out_type=(
            jax.ShapeDtypeStruct((NC, SEG_PAD, W), jnp.float32),
            jax.ShapeDtypeStruct((NC, ROWS_A, W), jnp.float32),
            jax.ShapeDtypeStruct((NXOUT,), jnp.int32),
        ),
        mesh=_mesh,
        scratch_types=[
            pltpu.VMEM((256,), jnp.int32),
            pltpu.VMEM((272,), jnp.int32),
            pltpu.VMEM((256,), jnp.int32),
            pltpu.VMEM((128,), jnp.int32),
            pltpu.VMEM((3, D_IN), jnp.float32),
            pltpu.VMEM((64, D_IN), jnp.float32),
            pltpu.VMEM((blk,), jnp.int32),
            pltpu.VMEM((blk,), jnp.int32),
            pltpu.VMEM((blk,), jnp.int32),
            pltpu.VMEM((blk,), jnp.int32),
            pltpu.VMEM((blk,), jnp.int32),
            pltpu.VMEM((blk,), jnp.int32),
            pltpu.VMEM((packs,), jnp.int32),
            pltpu.VMEM((packs // CH, CH), jnp.int32),
            pltpu.VMEM((CH, W), jnp.float32),
            pltpu.VMEM((CH, W), jnp.float32),
            pltpu.VMEM_SHARED((SEG_PAD, W), jnp.float32),
            pltpu.SemaphoreType.DMA,
            pltpu.SemaphoreType.DMA,
            pltpu.SemaphoreType.DMA,
            pltpu.SemaphoreType.DMA,
        ],
        compiler_params=_sc_params,
    )(functools.partial(_agg_kernel, eb, blk, True))


def _make_l1_call(eb, blk):
    packs = ((eb + CH - 1) // CH) * CH
    return functools.partial(
        pl.kernel,
        out_type=jax.ShapeDtypeStruct((NC, SEG_PAD, W), jnp.float32),
        mesh=_mesh,
        scratch_types=[
            pltpu.VMEM((blk,), jnp.int32),
            pltpu.VMEM((blk,), jnp.int32),
            pltpu.VMEM((blk,), jnp.int32),
            pltpu.VMEM((blk,), jnp.int32),
            pltpu.VMEM((blk,), jnp.int32),
            pltpu.VMEM((blk,), jnp.int32),
            pltpu.VMEM((packs,), jnp.int32),
            pltpu.VMEM((packs // CH, CH), jnp.int32),
            pltpu.VMEM((CH, W), jnp.float32),
            pltpu.VMEM((CH, W), jnp.float32),
            pltpu.VMEM_SHARED((SEG_PAD, W), jnp.float32),
            pltpu.SemaphoreType.DMA,
            pltpu.SemaphoreType.DMA,
            pltpu.SemaphoreType.DMA,
            pltpu.SemaphoreType.DMA,
        ],
        compiler_params=_sc_params,
    )(functools.partial(_agg_kernel, eb, blk, False))


def _dense0_kernel(acc_ref, xaug_ref, nt_ref, w_ref, rw_ref, rb_ref, out_ref):
    acc = acc_ref[0] + acc_ref[1]
    feats = acc[:NSEG, :D_IN]
    cnt = acc[:NSEG, D_IN:D_IN + 1]
    scaled = feats / jnp.maximum(cnt, 1.0)
    h = jnp.zeros((N2, D_HID), jnp.float32)
    for i in range(NUM_ET):
        h = h + jnp.dot(scaled[i * N2:(i + 1) * N2], w_ref[i].T,
                        preferred_element_type=jnp.float32)
    x_tgt = xaug_ref[:N2, :D_IN]
    nt = nt_ref[:N2]
    for t in range(NUM_NT):
        r = jnp.dot(x_tgt, rw_ref[t].T, preferred_element_type=jnp.float32)
        r = r + rb_ref[t][None, :]
        h = h + jnp.where(nt == t, r, 0.0)
    h = jnp.maximum(h, 0.0)
    out_ref[:, :D_IN] = h
    colpad = lax.broadcasted_iota(jnp.int32, (N2, W - D_IN), 1)
    out_ref[:, D_IN:] = jnp.where(colpad == 0, 1.0, 0.0)


def _dense1_kernel(acc_ref, xaug_ref, nt_ref, w_ref, rw_ref, rb_ref, out_ref):
    acc = acc_ref[0] + acc_ref[1]
    feats = acc[:NSEG, :D_HID]
    cnt = acc[:NSEG, D_HID:D_HID + 1]
    scaled = feats / jnp.maximum(cnt, 1.0)
    o = jnp.zeros((N2, D_OUT), jnp.float32)
    for i in range(NUM_ET):
        o = o + jnp.dot(scaled[i * N2:(i + 1) * N2], w_ref[i].T,
                        preferred_element_type=jnp.float32)
    x_tgt = xaug_ref[:N2, :D_HID]
    nt = nt_ref[:N2]
    for t in range(NUM_NT):
        r = jnp.dot(x_tgt, rw_ref[t].T, preferred_element_type=jnp.float32)
        r = r + rb_ref[t][None, :]
        o = o + jnp.where(nt == t, r, 0.0)
    m = jnp.max(o, axis=1, keepdims=True)
    e = jnp.exp(o - m)
    lse = jnp.log(jnp.sum(e, axis=1, keepdims=True))
    out_ref[...] = o - m - lse


def _pad_edges(src, dst, et, epad):
    e = src.shape[0]
    pad = epad - e
    src = jnp.concatenate([src.astype(jnp.int32), jnp.zeros((pad,), jnp.int32)])
    dst = jnp.concatenate([dst.astype(jnp.int32),
                           jnp.full((pad,), N1, jnp.int32)])
    et = jnp.concatenate([et.astype(jnp.int32), jnp.zeros((pad,), jnp.int32)])
    return src, dst, et


def kernel(n_id, x_paper, edge_index_0, edge_type_0, edge_index_1, edge_type_1,
           node_type, local_node_idx, emb, rel_W0, root_W0, root_b0,
           rel_W1, root_W1, root_b1):
    src0, dst0, et0 = _pad_edges(edge_index_0[0], edge_index_0[1],
                                 edge_type_0, 262144)
    acc0, xaug, nt1024 = _make_l0_call(262144 // NW, BLK)(
        x_paper, emb, src0, dst0, et0, n_id.astype(jnp.int32),
        node_type.astype(jnp.int32), local_node_idx.astype(jnp.int32))

    nt1000 = nt1024[:N2][:, None]
    h1aug = pl.pallas_call(
        _dense0_kernel,
        out_shape=jax.ShapeDtypeStruct((N2, W), jnp.float32),
    )(acc0, xaug[0, :N2], nt1000, rel_W0, root_W0, root_b0)

    src1, dst1, et1 = _pad_edges(edge_index_1[0], edge_index_1[1],
                                 edge_type_1, 65536)
    acc1 = _make_l1_call(65536 // NW, BLK)(h1aug, src1, dst1, et1)

    out = pl.pallas_call(
        _dense1_kernel,
        out_shape=jax.ShapeDtypeStruct((N2, D_OUT), jnp.float32),
    )(acc1, h1aug, nt1000, rel_W1, root_W1, root_b1)
    return out
